# Initial kernel scaffold; baseline (speedup 1.0000x reference)
import functools

import jax
import jax.numpy as jnp
from jax import lax
from jax.experimental import pallas as pl
from jax.experimental.pallas import tpu as pltpu
from jax.experimental.pallas import tpu_sc as plsc

_N = 10000
_NP = 10240
_NC = 2
_NS = 16
_K = 128
_G = 8
_ROWS_PER_SUB = _NP // _NS

_BN_EPS = 1e-5
_SELU_SCALE = 1.0507009873554805
_SELU_ALPHA = 1.6732632423543772


def _make_prop(f, e_pad):
    e_per_worker = e_pad // (_NC * _NS)
    rows_per_worker = e_per_worker // _K
    n_groups = rows_per_worker // _G
    assert rows_per_worker % _G == 0

    mesh = plsc.VectorSubcoreMesh(core_axis_name="c", subcore_axis_name="s")

    @functools.partial(
        pl.kernel,
        out_type=jax.ShapeDtypeStruct((_NC, _NP, f), jnp.float32),
        mesh=mesh,
        scratch_types=[
            pltpu.VMEM_SHARED((_NP, f), jnp.float32),
            pltpu.VMEM((_G, _K), jnp.int32),
            pltpu.VMEM((_G, _K), jnp.int32),
            pltpu.VMEM((_K, f), jnp.float32),
            pltpu.SemaphoreType.DMA,
        ],
    )
    def prop(h_hbm, src_hbm, dst_hbm, zeros_hbm, out_hbm, acc, src_v, dst_v,
             rows_v, sem):
        c = lax.axis_index("c")
        s = lax.axis_index("s")
        row0 = s * _ROWS_PER_SUB

        @pl.when(c == 0)
        def _():
            pltpu.sync_copy(h_hbm.at[pl.ds(row0, _ROWS_PER_SUB)],
                            acc.at[pl.ds(row0, _ROWS_PER_SUB)])

        @pl.when(c != 0)
        def _():
            pltpu.sync_copy(zeros_hbm.at[pl.ds(row0, _ROWS_PER_SUB)],
                            acc.at[pl.ds(row0, _ROWS_PER_SUB)])

        plsc.subcore_barrier()

        wid = c * _NS + s
        base_row = wid * rows_per_worker

        @pl.loop(0, n_groups)
        def _(g):
            r = base_row + g * _G
            pltpu.sync_copy(src_hbm.at[pl.ds(r, _G)], src_v)
            pltpu.sync_copy(dst_hbm.at[pl.ds(r, _G)], dst_v)
            for j in range(_G):
                pltpu.async_copy(h_hbm.at[src_v.at[j]], rows_v, sem).wait()
                pltpu.sync_copy(rows_v, acc.at[dst_v.at[j]], add=True)

        plsc.subcore_barrier()
        pltpu.sync_copy(acc.at[pl.ds(row0, _ROWS_PER_SUB)],
                        out_hbm.at[c].at[pl.ds(row0, _ROWS_PER_SUB)])

    return prop


def _mm1(x, w1):
    def body(x_ref, w_ref, o_ref):
        o_ref[...] = jnp.dot(x_ref[...], w_ref[...],
                             preferred_element_type=jnp.float32)

    return pl.pallas_call(
        body,
        out_shape=jax.ShapeDtypeStruct((x.shape[0], w1.shape[1]), jnp.float32),
    )(x, w1)


def _add_parts(p):
    def body(p_ref, o_ref):
        o_ref[...] = p_ref[0] + p_ref[1]

    return pl.pallas_call(
        body,
        out_shape=jax.ShapeDtypeStruct(p.shape[1:], jnp.float32),
    )(p)


def _mid(p, gamma, beta, w2):
    npad = p.shape[1]

    def body(p_ref, g_ref, b_ref, w_ref, o_ref):
        z = p_ref[0] + p_ref[1]
        rows = lax.broadcasted_iota(jnp.int32, (npad, 1), 0)
        mask = (rows < _N).astype(jnp.float32)
        zm = z * mask
        mean = jnp.sum(zm, axis=0, keepdims=True) * (1.0 / _N)
        d = (z - mean) * mask
        var = jnp.sum(d * d, axis=0, keepdims=True) * (1.0 / _N)
        hn = (z - mean) * jax.lax.rsqrt(var + _BN_EPS) * g_ref[...] + b_ref[...]
        hs = _SELU_SCALE * jnp.where(hn > 0, hn,
                                     _SELU_ALPHA * (jnp.exp(hn) - 1.0))
        o_ref[...] = jnp.dot(hs, w_ref[...],
                             preferred_element_type=jnp.float32)

    return pl.pallas_call(
        body,
        out_shape=jax.ShapeDtypeStruct((npad, w2.shape[1]), jnp.float32),
    )(p, gamma, beta, w2)


def _final(p, b2):
    npad, cdim = p.shape[1], p.shape[2]

    def body(p_ref, b_ref, o_ref):
        z = p_ref[0] + p_ref[1] + b_ref[...]
        m = jnp.max(z, axis=1, keepdims=True)
        lse = jnp.log(jnp.sum(jnp.exp(z - m), axis=1, keepdims=True)) + m
        o_ref[...] = z - lse

    return pl.pallas_call(
        body,
        out_shape=jax.ShapeDtypeStruct((npad, cdim), jnp.float32),
    )(p, b2)


@jax.jit
def kernel(x, edge_index, W1, b1, gamma, beta, W2, b2):
    del b1
    n, _ = x.shape
    e = edge_index.shape[1]
    quantum = _NC * _NS * _K * _G
    e_pad = ((e + quantum - 1) // quantum) * quantum

    src = edge_index[0].astype(jnp.int32)
    dst = edge_index[1].astype(jnp.int32)
    pad_e = e_pad - e
    dump = _N + 224 + (jnp.arange(pad_e, dtype=jnp.int32) % 16)
    src_p = jnp.concatenate([src, jnp.zeros((pad_e,), jnp.int32)]).reshape(-1, _K)
    dst_p = jnp.concatenate([dst, dump]).reshape(-1, _K)

    x_p = jnp.pad(x, ((0, _NP - n), (0, 0)))
    zeros_h = jnp.zeros((_NP, W1.shape[1]), jnp.float32)
    zeros_c = jnp.zeros((_NP, W2.shape[1]), jnp.float32)

    prop_h = _make_prop(W1.shape[1], e_pad)
    prop_c = _make_prop(W2.shape[1], e_pad)

    y1 = _mm1(x_p, W1)
    p1 = prop_h(y1, src_p, dst_p, zeros_h)
    h1 = _add_parts(p1)
    p2 = prop_h(h1, src_p, dst_p, zeros_h)
    y2 = _mid(p2, gamma.reshape(1, -1), beta.reshape(1, -1), W2)
    p3 = prop_c(y2, src_p, dst_p, zeros_c)
    out = _final(p3, b2.reshape(1, -1))
    return out[:n]


# speedup vs baseline: 5.0224x; 5.0224x over previous
import functools

import jax
import jax.numpy as jnp
from jax import lax
from jax.experimental import pallas as pl
from jax.experimental.pallas import tpu as pltpu
from jax.experimental.pallas import tpu_sc as plsc

_N = 10000
_NP = 10240
_NC = 2
_NS = 16
_K = 128
_G = 8
_ROWS_PER_SUB = _NP // _NS

_BN_EPS = 1e-5
_SELU_SCALE = 1.0507009873554805
_SELU_ALPHA = 1.6732632423543772


def _make_prop(f, e_pad):
    e_per_worker = e_pad // (_NC * _NS)
    rows_per_worker = e_per_worker // _K
    n_groups = rows_per_worker // _G
    assert rows_per_worker % _G == 0

    mesh = plsc.VectorSubcoreMesh(core_axis_name="c", subcore_axis_name="s",
                                  num_cores=_NC, num_subcores=_NS)

    @functools.partial(
        pl.kernel,
        out_type=jax.ShapeDtypeStruct((_NC, _NP, f), jnp.float32),
        mesh=mesh,
        scratch_types=[
            pltpu.VMEM_SHARED((_NP, f), jnp.float32),
            pltpu.VMEM((_G, _K), jnp.int32),
            pltpu.VMEM((_G, _K), jnp.int32),
            pltpu.VMEM((_K, f), jnp.float32),
            pltpu.SemaphoreType.DMA,
        ],
        compiler_params=pltpu.CompilerParams(use_tc_tiling_on_sc=False),
    )
    def prop(h_hbm, src_hbm, dst_hbm, zeros_hbm, out_hbm, acc, src_v, dst_v,
             rows_v, sem):
        c = lax.axis_index("c")
        s = lax.axis_index("s")
        row0 = s * _ROWS_PER_SUB

        @pl.when(c == 0)
        def _():
            pltpu.sync_copy(h_hbm.at[pl.ds(row0, _ROWS_PER_SUB)],
                            acc.at[pl.ds(row0, _ROWS_PER_SUB)])

        @pl.when(c != 0)
        def _():
            pltpu.sync_copy(zeros_hbm.at[pl.ds(row0, _ROWS_PER_SUB)],
                            acc.at[pl.ds(row0, _ROWS_PER_SUB)])

        plsc.subcore_barrier()

        wid = c * _NS + s
        base_row = wid * rows_per_worker

        @pl.loop(0, n_groups)
        def _(g):
            r = base_row + g * _G
            pltpu.sync_copy(src_hbm.at[pl.ds(r, _G)], src_v)
            pltpu.sync_copy(dst_hbm.at[pl.ds(r, _G)], dst_v)
            for j in range(_G):
                pltpu.async_copy(h_hbm.at[src_v.at[j]], rows_v, sem).wait()
                pltpu.sync_copy(rows_v, acc.at[dst_v.at[j]], add=True)

        plsc.subcore_barrier()
        pltpu.sync_copy(acc.at[pl.ds(row0, _ROWS_PER_SUB)],
                        out_hbm.at[c].at[pl.ds(row0, _ROWS_PER_SUB)])

    return prop


def _mm1(x, w1):
    def body(x_ref, w_ref, o_ref):
        o_ref[...] = jnp.dot(x_ref[...], w_ref[...],
                             preferred_element_type=jnp.float32)

    return pl.pallas_call(
        body,
        out_shape=jax.ShapeDtypeStruct((x.shape[0], w1.shape[1]), jnp.float32),
    )(x, w1)


def _add_parts(p):
    def body(p_ref, o_ref):
        o_ref[...] = p_ref[0] + p_ref[1]

    return pl.pallas_call(
        body,
        out_shape=jax.ShapeDtypeStruct(p.shape[1:], jnp.float32),
    )(p)


def _mid(p, gamma, beta, w2):
    npad = p.shape[1]

    def body(p_ref, g_ref, b_ref, w_ref, o_ref):
        z = p_ref[0] + p_ref[1]
        rows = lax.broadcasted_iota(jnp.int32, (npad, 1), 0)
        mask = (rows < _N).astype(jnp.float32)
        zm = z * mask
        mean = jnp.sum(zm, axis=0, keepdims=True) * (1.0 / _N)
        d = (z - mean) * mask
        var = jnp.sum(d * d, axis=0, keepdims=True) * (1.0 / _N)
        hn = (z - mean) * jax.lax.rsqrt(var + _BN_EPS) * g_ref[...] + b_ref[...]
        hs = _SELU_SCALE * jnp.where(hn > 0, hn,
                                     _SELU_ALPHA * (jnp.exp(hn) - 1.0))
        o_ref[...] = jnp.dot(hs, w_ref[...],
                             preferred_element_type=jnp.float32)

    return pl.pallas_call(
        body,
        out_shape=jax.ShapeDtypeStruct((npad, w2.shape[1]), jnp.float32),
    )(p, gamma, beta, w2)


def _final(p, b2):
    npad, cdim = p.shape[1], p.shape[2]

    def body(p_ref, b_ref, o_ref):
        z = p_ref[0] + p_ref[1] + b_ref[...]
        m = jnp.max(z, axis=1, keepdims=True)
        lse = jnp.log(jnp.sum(jnp.exp(z - m), axis=1, keepdims=True)) + m
        o_ref[...] = z - lse

    return pl.pallas_call(
        body,
        out_shape=jax.ShapeDtypeStruct((npad, cdim), jnp.float32),
    )(p, b2)


@jax.jit
def kernel(x, edge_index, W1, b1, gamma, beta, W2, b2):
    del b1
    n, _ = x.shape
    e = edge_index.shape[1]
    quantum = _NC * _NS * _K * _G
    e_pad = ((e + quantum - 1) // quantum) * quantum

    src = edge_index[0].astype(jnp.int32)
    dst = edge_index[1].astype(jnp.int32)
    pad_e = e_pad - e
    dump = _N + 224 + (jnp.arange(pad_e, dtype=jnp.int32) % 16)
    src_p = jnp.concatenate([src, jnp.zeros((pad_e,), jnp.int32)]).reshape(-1, _K)
    dst_p = jnp.concatenate([dst, dump]).reshape(-1, _K)

    x_p = jnp.pad(x, ((0, _NP - n), (0, 0)))
    zeros_h = jnp.zeros((_NP, W1.shape[1]), jnp.float32)
    zeros_c = jnp.zeros((_NP, W2.shape[1]), jnp.float32)

    prop_h = _make_prop(W1.shape[1], e_pad)
    prop_c = _make_prop(W2.shape[1], e_pad)

    y1 = _mm1(x_p, W1)
    p1 = prop_h(y1, src_p, dst_p, zeros_h)
    h1 = _add_parts(p1)
    p2 = prop_h(h1, src_p, dst_p, zeros_h)
    y2 = _mid(p2, gamma.reshape(1, -1), beta.reshape(1, -1), W2)
    p3 = prop_c(y2, src_p, dst_p, zeros_c)
    out = _final(p3, b2.reshape(1, -1))
    return out[:n]


# speedup vs baseline: 6.0383x; 1.2023x over previous
import functools

import jax
import jax.numpy as jnp
from jax import lax
from jax.experimental import pallas as pl
from jax.experimental.pallas import tpu as pltpu
from jax.experimental.pallas import tpu_sc as plsc

_N = 10000
_NP = 10240
_NC = 2
_NS = 16
_K = 128
_G = 4
_ROWS_PER_SUB = _NP // _NS

_BN_EPS = 1e-5
_SELU_SCALE = 1.0507009873554805
_SELU_ALPHA = 1.6732632423543772


def _make_prop(f, e_pad):
    e_per_worker = e_pad // (_NC * _NS)
    rows_per_worker = e_per_worker // _K
    n_groups = rows_per_worker // _G
    assert rows_per_worker % _G == 0
    bank_rows = _G * _K

    mesh = plsc.VectorSubcoreMesh(core_axis_name="c", subcore_axis_name="s",
                                  num_cores=_NC, num_subcores=_NS)

    @functools.partial(
        pl.kernel,
        out_type=jax.ShapeDtypeStruct((_NC, _NP, f), jnp.float32),
        mesh=mesh,
        scratch_types=[
            pltpu.VMEM_SHARED((_NP, f), jnp.float32),
            pltpu.VMEM((rows_per_worker, _K), jnp.int32),
            pltpu.VMEM((rows_per_worker, _K), jnp.int32),
            pltpu.VMEM((bank_rows, f), jnp.float32),
            pltpu.VMEM((bank_rows, f), jnp.float32),
            pltpu.SemaphoreType.DMA,
            pltpu.SemaphoreType.DMA,
            pltpu.SemaphoreType.DMA,
            pltpu.SemaphoreType.DMA,
            pltpu.SemaphoreType.DMA,
        ],
        compiler_params=pltpu.CompilerParams(use_tc_tiling_on_sc=False),
    )
    def prop(h_hbm, src_hbm, dst_hbm, zeros_hbm, out_hbm, acc, src_v,
             dst_v, rows0, rows1, sem0, semg0, semg1, sems0, sems1):
        c = lax.axis_index("c")
        s = lax.axis_index("s")
        row0 = s * _ROWS_PER_SUB
        wid = c * _NS + s
        base_row = wid * rows_per_worker
        nsl = pl.ds(row0, _ROWS_PER_SUB)

        pltpu.async_copy(src_hbm.at[pl.ds(base_row, rows_per_worker)], src_v,
                         sem0)
        pltpu.async_copy(dst_hbm.at[pl.ds(base_row, rows_per_worker)], dst_v,
                         sem0)

        @pl.when(c == 0)
        def _():
            pltpu.async_copy(h_hbm.at[nsl], acc.at[nsl], sem0).wait()

        @pl.when(c != 0)
        def _():
            pltpu.async_copy(zeros_hbm.at[nsl], acc.at[nsl], sem0).wait()

        pltpu.make_async_copy(
            src_hbm.at[pl.ds(base_row, rows_per_worker)], src_v, sem0).wait()
        pltpu.make_async_copy(
            dst_hbm.at[pl.ds(base_row, rows_per_worker)], dst_v, sem0).wait()
        plsc.subcore_barrier()

        banks = ((rows0, semg0, sems0), (rows1, semg1, sems1))

        def gather_desc(g, rows_b, semg, j):
            return pltpu.make_async_copy(h_hbm.at[src_v.at[g * _G + j]],
                                         rows_b.at[pl.ds(j * _K, _K)], semg)

        def scatter_desc(g, rows_b, sems, j):
            return pltpu.make_async_copy(rows_b.at[pl.ds(j * _K, _K)],
                                         acc.at[dst_v.at[g * _G + j]], sems)

        @pl.loop(0, n_groups, step=2)
        def _(g0):
            for bi in range(2):
                rows_b, semg, sems = banks[bi]
                g = g0 + bi

                @pl.when(g0 >= 2)
                def _():
                    for j in range(_G):
                        scatter_desc(g - 2, rows_b, sems, j).wait()

                for j in range(_G):
                    gather_desc(g, rows_b, semg, j).start()
                for j in range(_G):
                    gather_desc(g, rows_b, semg, j).wait()
                for j in range(_G):
                    scatter_desc(g, rows_b, sems, j).start(add=True)

        for bi in range(2):
            rows_b, _, sems = banks[bi]
            g = n_groups - 2 + bi
            for j in range(_G):
                scatter_desc(g, rows_b, sems, j).wait()

        plsc.subcore_barrier()
        pltpu.sync_copy(acc.at[nsl], out_hbm.at[c].at[nsl])

    return prop


def _mm1(x, w1):
    def body(x_ref, w_ref, o_ref):
        o_ref[...] = jnp.dot(x_ref[...], w_ref[...],
                             preferred_element_type=jnp.float32)

    return pl.pallas_call(
        body,
        out_shape=jax.ShapeDtypeStruct((x.shape[0], w1.shape[1]), jnp.float32),
    )(x, w1)


def _add_parts(p):
    def body(p_ref, o_ref):
        o_ref[...] = p_ref[0] + p_ref[1]

    return pl.pallas_call(
        body,
        out_shape=jax.ShapeDtypeStruct(p.shape[1:], jnp.float32),
    )(p)


def _mid(p, gamma, beta, w2):
    npad = p.shape[1]

    def body(p_ref, g_ref, b_ref, w_ref, o_ref):
        z = p_ref[0] + p_ref[1]
        rows = lax.broadcasted_iota(jnp.int32, (npad, 1), 0)
        mask = (rows < _N).astype(jnp.float32)
        zm = z * mask
        mean = jnp.sum(zm, axis=0, keepdims=True) * (1.0 / _N)
        d = (z - mean) * mask
        var = jnp.sum(d * d, axis=0, keepdims=True) * (1.0 / _N)
        hn = (z - mean) * jax.lax.rsqrt(var + _BN_EPS) * g_ref[...] + b_ref[...]
        hs = _SELU_SCALE * jnp.where(hn > 0, hn,
                                     _SELU_ALPHA * (jnp.exp(hn) - 1.0))
        o_ref[...] = jnp.dot(hs, w_ref[...],
                             preferred_element_type=jnp.float32)

    return pl.pallas_call(
        body,
        out_shape=jax.ShapeDtypeStruct((npad, w2.shape[1]), jnp.float32),
    )(p, gamma, beta, w2)


def _final(p, b2):
    npad, cdim = p.shape[1], p.shape[2]

    def body(p_ref, b_ref, o_ref):
        z = p_ref[0] + p_ref[1] + b_ref[...]
        m = jnp.max(z, axis=1, keepdims=True)
        lse = jnp.log(jnp.sum(jnp.exp(z - m), axis=1, keepdims=True)) + m
        o_ref[...] = z - lse

    return pl.pallas_call(
        body,
        out_shape=jax.ShapeDtypeStruct((npad, cdim), jnp.float32),
    )(p, b2)


@jax.jit
def kernel(x, edge_index, W1, b1, gamma, beta, W2, b2):
    del b1
    n, _ = x.shape
    e = edge_index.shape[1]
    quantum = _NC * _NS * _K * _G
    e_pad = ((e + quantum - 1) // quantum) * quantum

    src = edge_index[0].astype(jnp.int32)
    dst = edge_index[1].astype(jnp.int32)
    pad_e = e_pad - e
    dump = _N + (jnp.arange(pad_e, dtype=jnp.int32) % (_NP - _N))
    src_p = jnp.concatenate([src, jnp.zeros((pad_e,), jnp.int32)]).reshape(-1, _K)
    dst_p = jnp.concatenate([dst, dump]).reshape(-1, _K)

    x_p = jnp.pad(x, ((0, _NP - n), (0, 0)))
    zeros_h = jnp.zeros((_NP, W1.shape[1]), jnp.float32)
    zeros_c = jnp.zeros((_NP, W2.shape[1]), jnp.float32)

    prop_h = _make_prop(W1.shape[1], e_pad)
    prop_c = _make_prop(W2.shape[1], e_pad)

    y1 = _mm1(x_p, W1)
    p1 = prop_h(y1, src_p, dst_p, zeros_h)
    h1 = _add_parts(p1)
    p2 = prop_h(h1, src_p, dst_p, zeros_h)
    y2 = _mid(p2, gamma.reshape(1, -1), beta.reshape(1, -1), W2)
    p3 = prop_c(y2, src_p, dst_p, zeros_c)
    out = _final(p3, b2.reshape(1, -1))
    return out[:n]


# speedup vs baseline: 6.1721x; 1.0222x over previous
import functools

import jax
import jax.numpy as jnp
from jax import lax
from jax.experimental import pallas as pl
from jax.experimental.pallas import tpu as pltpu
from jax.experimental.pallas import tpu_sc as plsc

_N = 10000
_NP = 10240
_NC = 2
_NS = 16
_K = 128
_G = 4
_ROWS_PER_SUB = _NP // _NS

_BN_EPS = 1e-5
_SELU_SCALE = 1.0507009873554805
_SELU_ALPHA = 1.6732632423543772


def _make_prop(f, e_pad):
    e_per_worker = e_pad // (_NC * _NS)
    rows_per_worker = e_per_worker // _K
    n_groups = rows_per_worker // _G
    assert rows_per_worker % _G == 0
    bank_rows = _G * _K

    mesh = plsc.VectorSubcoreMesh(core_axis_name="c", subcore_axis_name="s",
                                  num_cores=_NC, num_subcores=_NS)

    @functools.partial(
        pl.kernel,
        out_type=jax.ShapeDtypeStruct((_NC, _NP, f), jnp.float32),
        mesh=mesh,
        scratch_types=[
            pltpu.VMEM_SHARED((_NP, f), jnp.float32),
            pltpu.VMEM((rows_per_worker, _K), jnp.int32),
            pltpu.VMEM((rows_per_worker, _K), jnp.int32),
            pltpu.VMEM((bank_rows, f), jnp.float32),
            pltpu.VMEM((bank_rows, f), jnp.float32),
            pltpu.SemaphoreType.DMA,
            pltpu.SemaphoreType.DMA,
            pltpu.SemaphoreType.DMA,
            pltpu.SemaphoreType.DMA,
            pltpu.SemaphoreType.DMA,
        ],
        compiler_params=pltpu.CompilerParams(use_tc_tiling_on_sc=False),
    )
    def prop(h_hbm, src_hbm, dst_hbm, zeros_hbm, out_hbm, acc, src_v,
             dst_v, rows0, rows1, sem0, semg0, semg1, sems0, sems1):
        c = lax.axis_index("c")
        s = lax.axis_index("s")
        row0 = s * _ROWS_PER_SUB
        wid = c * _NS + s
        base_row = wid * rows_per_worker
        nsl = pl.ds(row0, _ROWS_PER_SUB)

        pltpu.async_copy(src_hbm.at[pl.ds(base_row, rows_per_worker)], src_v,
                         sem0)
        pltpu.async_copy(dst_hbm.at[pl.ds(base_row, rows_per_worker)], dst_v,
                         sem0)

        @pl.when(c == 0)
        def _():
            pltpu.async_copy(h_hbm.at[nsl], acc.at[nsl], sem0).wait()

        @pl.when(c != 0)
        def _():
            pltpu.async_copy(zeros_hbm.at[nsl], acc.at[nsl], sem0).wait()

        pltpu.make_async_copy(
            src_hbm.at[pl.ds(base_row, rows_per_worker)], src_v, sem0).wait()
        pltpu.make_async_copy(
            dst_hbm.at[pl.ds(base_row, rows_per_worker)], dst_v, sem0).wait()
        plsc.subcore_barrier()

        banks = ((rows0, semg0, sems0), (rows1, semg1, sems1))

        def gather_desc(g, rows_b, semg, j):
            return pltpu.make_async_copy(h_hbm.at[src_v.at[g * _G + j]],
                                         rows_b.at[pl.ds(j * _K, _K)], semg)

        def scatter_desc(g, rows_b, sems, j):
            return pltpu.make_async_copy(rows_b.at[pl.ds(j * _K, _K)],
                                         acc.at[dst_v.at[g * _G + j]], sems)

        rows_p, semg_p, _ = banks[0]
        for j in range(_G):
            gather_desc(0, rows_p, semg_p, j).start()

        @pl.loop(0, n_groups, step=2)
        def _(g0):
            for bi in range(2):
                rows_c, semg_c, sems_c = banks[bi]
                rows_n, semg_n, sems_n = banks[1 - bi]
                g = g0 + bi

                @pl.when(g >= 1)
                def _():
                    for j in range(_G):
                        scatter_desc(g - 1, rows_n, sems_n, j).wait()

                @pl.when(g + 1 < n_groups)
                def _():
                    for j in range(_G):
                        gather_desc(g + 1, rows_n, semg_n, j).start()

                for j in range(_G):
                    gather_desc(g, rows_c, semg_c, j).wait()
                for j in range(_G):
                    scatter_desc(g, rows_c, sems_c, j).start(add=True)

        g_last = n_groups - 1
        rows_b, _, sems = banks[g_last % 2]
        for j in range(_G):
            scatter_desc(g_last, rows_b, sems, j).wait()

        plsc.subcore_barrier()
        pltpu.sync_copy(acc.at[nsl], out_hbm.at[c].at[nsl])

    return prop


def _mm1(x, w1):
    def body(x_ref, w_ref, o_ref):
        o_ref[...] = jnp.dot(x_ref[...], w_ref[...],
                             preferred_element_type=jnp.float32)

    return pl.pallas_call(
        body,
        out_shape=jax.ShapeDtypeStruct((x.shape[0], w1.shape[1]), jnp.float32),
    )(x, w1)


def _add_parts(p):
    def body(p_ref, o_ref):
        o_ref[...] = p_ref[0] + p_ref[1]

    return pl.pallas_call(
        body,
        out_shape=jax.ShapeDtypeStruct(p.shape[1:], jnp.float32),
    )(p)


def _mid(p, gamma, beta, w2):
    npad = p.shape[1]

    def body(p_ref, g_ref, b_ref, w_ref, o_ref):
        z = p_ref[0] + p_ref[1]
        rows = lax.broadcasted_iota(jnp.int32, (npad, 1), 0)
        mask = (rows < _N).astype(jnp.float32)
        zm = z * mask
        mean = jnp.sum(zm, axis=0, keepdims=True) * (1.0 / _N)
        d = (z - mean) * mask
        var = jnp.sum(d * d, axis=0, keepdims=True) * (1.0 / _N)
        hn = (z - mean) * jax.lax.rsqrt(var + _BN_EPS) * g_ref[...] + b_ref[...]
        hs = _SELU_SCALE * jnp.where(hn > 0, hn,
                                     _SELU_ALPHA * (jnp.exp(hn) - 1.0))
        o_ref[...] = jnp.dot(hs, w_ref[...],
                             preferred_element_type=jnp.float32)

    return pl.pallas_call(
        body,
        out_shape=jax.ShapeDtypeStruct((npad, w2.shape[1]), jnp.float32),
    )(p, gamma, beta, w2)


def _final(p, b2):
    npad, cdim = p.shape[1], p.shape[2]

    def body(p_ref, b_ref, o_ref):
        z = p_ref[0] + p_ref[1] + b_ref[...]
        m = jnp.max(z, axis=1, keepdims=True)
        lse = jnp.log(jnp.sum(jnp.exp(z - m), axis=1, keepdims=True)) + m
        o_ref[...] = z - lse

    return pl.pallas_call(
        body,
        out_shape=jax.ShapeDtypeStruct((npad, cdim), jnp.float32),
    )(p, b2)


@jax.jit
def kernel(x, edge_index, W1, b1, gamma, beta, W2, b2):
    del b1
    n, _ = x.shape
    e = edge_index.shape[1]
    quantum = _NC * _NS * _K * _G
    e_pad = ((e + quantum - 1) // quantum) * quantum

    src = edge_index[0].astype(jnp.int32)
    dst = edge_index[1].astype(jnp.int32)
    pad_e = e_pad - e
    dump = _N + (jnp.arange(pad_e, dtype=jnp.int32) % (_NP - _N))
    src_p = jnp.concatenate([src, jnp.zeros((pad_e,), jnp.int32)]).reshape(-1, _K)
    dst_p = jnp.concatenate([dst, dump]).reshape(-1, _K)

    x_p = jnp.pad(x, ((0, _NP - n), (0, 0)))
    zeros_h = jnp.zeros((_NP, W1.shape[1]), jnp.float32)
    zeros_c = jnp.zeros((_NP, W2.shape[1]), jnp.float32)

    prop_h = _make_prop(W1.shape[1], e_pad)
    prop_c = _make_prop(W2.shape[1], e_pad)

    y1 = _mm1(x_p, W1)
    p1 = prop_h(y1, src_p, dst_p, zeros_h)
    h1 = _add_parts(p1)
    p2 = prop_h(h1, src_p, dst_p, zeros_h)
    y2 = _mid(p2, gamma.reshape(1, -1), beta.reshape(1, -1), W2)
    p3 = prop_c(y2, src_p, dst_p, zeros_c)
    out = _final(p3, b2.reshape(1, -1))
    return out[:n]


# speedup vs baseline: 14.5821x; 2.3626x over previous
import functools

import jax
import jax.numpy as jnp
from jax import lax
from jax.experimental import pallas as pl
from jax.experimental.pallas import tpu as pltpu
from jax.experimental.pallas import tpu_sc as plsc

_N = 10000
_NP = 10240
_NC = 2
_NS = 16
_K = 128
_G = 4
_ROWS_PER_SUB = _NP // _NS

_BN_EPS = 1e-5
_SELU_SCALE = 1.0507009873554805
_SELU_ALPHA = 1.6732632423543772


def _make_prop(fh, e_pad):
    e_per_worker = e_pad // _NS
    rows_per_worker = e_per_worker // _K
    n_groups = rows_per_worker // _G
    assert rows_per_worker % _G == 0 and n_groups % 2 == 0
    bank_rows = _G * _K

    mesh = plsc.VectorSubcoreMesh(core_axis_name="c", subcore_axis_name="s",
                                  num_cores=_NC, num_subcores=_NS)

    @functools.partial(
        pl.kernel,
        out_type=jax.ShapeDtypeStruct((_NC, _NP, fh), jnp.float32),
        mesh=mesh,
        scratch_types=[
            pltpu.VMEM_SHARED((_NP, fh), jnp.float32),
            pltpu.VMEM_SHARED((_NP, fh), jnp.float32),
            pltpu.VMEM((rows_per_worker, _K), jnp.int32),
            pltpu.VMEM((rows_per_worker, _K), jnp.int32),
            pltpu.VMEM((bank_rows, fh), jnp.float32),
            pltpu.VMEM((bank_rows, fh), jnp.float32),
            pltpu.SemaphoreType.DMA,
            pltpu.SemaphoreType.DMA,
            pltpu.SemaphoreType.DMA,
            pltpu.SemaphoreType.DMA,
            pltpu.SemaphoreType.DMA,
        ],
        compiler_params=pltpu.CompilerParams(use_tc_tiling_on_sc=False),
    )
    def prop(h_hbm, src_hbm, dst_hbm, out_hbm, acc, h_st, src_v, dst_v,
             rows0, rows1, sem0, semg0, semg1, sems0, sems1):
        c = lax.axis_index("c")
        s = lax.axis_index("s")
        row0 = s * _ROWS_PER_SUB
        base_row = s * rows_per_worker
        nsl = pl.ds(row0, _ROWS_PER_SUB)

        pltpu.async_copy(h_hbm.at[c].at[nsl], h_st.at[nsl], sem0)
        pltpu.async_copy(h_hbm.at[c].at[nsl], acc.at[nsl], sem0)
        pltpu.async_copy(src_hbm.at[pl.ds(base_row, rows_per_worker)], src_v,
                         sem0)
        pltpu.async_copy(dst_hbm.at[pl.ds(base_row, rows_per_worker)], dst_v,
                         sem0)
        pltpu.make_async_copy(h_hbm.at[c].at[nsl], h_st.at[nsl], sem0).wait()
        pltpu.make_async_copy(h_hbm.at[c].at[nsl], acc.at[nsl], sem0).wait()
        pltpu.make_async_copy(
            src_hbm.at[pl.ds(base_row, rows_per_worker)], src_v, sem0).wait()
        pltpu.make_async_copy(
            dst_hbm.at[pl.ds(base_row, rows_per_worker)], dst_v, sem0).wait()
        plsc.subcore_barrier()

        banks = ((rows0, semg0, sems0), (rows1, semg1, sems1))

        def gather_desc(g, rows_b, semg, j):
            return pltpu.make_async_copy(h_st.at[src_v.at[g * _G + j]],
                                         rows_b.at[pl.ds(j * _K, _K)], semg)

        def scatter_desc(g, rows_b, sems, j):
            return pltpu.make_async_copy(rows_b.at[pl.ds(j * _K, _K)],
                                         acc.at[dst_v.at[g * _G + j]], sems)

        rows_p, semg_p, _ = banks[0]
        for j in range(_G):
            gather_desc(0, rows_p, semg_p, j).start()

        @pl.loop(0, n_groups, step=2)
        def _(g0):
            for bi in range(2):
                rows_c, semg_c, sems_c = banks[bi]
                rows_n, semg_n, sems_n = banks[1 - bi]
                g = g0 + bi

                @pl.when(g >= 1)
                def _():
                    for j in range(_G):
                        scatter_desc(g - 1, rows_n, sems_n, j).wait()

                @pl.when(g + 1 < n_groups)
                def _():
                    for j in range(_G):
                        gather_desc(g + 1, rows_n, semg_n, j).start()

                for j in range(_G):
                    gather_desc(g, rows_c, semg_c, j).wait()
                for j in range(_G):
                    scatter_desc(g, rows_c, sems_c, j).start(add=True)

        g_last = n_groups - 1
        rows_b, _, sems = banks[g_last % 2]
        for j in range(_G):
            scatter_desc(g_last, rows_b, sems, j).wait()

        plsc.subcore_barrier()
        pltpu.sync_copy(acc.at[nsl], out_hbm.at[c].at[nsl])

    return prop


def _mm1(x, w1):
    npad = x.shape[0]
    h = w1.shape[1]
    fh = h // 2

    def body(x_ref, w_ref, o_ref):
        y = jnp.dot(x_ref[...], w_ref[...], preferred_element_type=jnp.float32)
        o_ref[0] = y[:, :fh]
        o_ref[1] = y[:, fh:]

    return pl.pallas_call(
        body,
        out_shape=jax.ShapeDtypeStruct((2, npad, fh), jnp.float32),
    )(x, w1)


def _mid(p, gamma, beta, w2):
    npad = p.shape[1]
    ch = w2.shape[1] // 2

    def body(p_ref, g_ref, b_ref, w_ref, o_ref):
        z = jnp.concatenate([p_ref[0], p_ref[1]], axis=1)
        rows = lax.broadcasted_iota(jnp.int32, (npad, 1), 0)
        mask = (rows < _N).astype(jnp.float32)
        zm = z * mask
        mean = jnp.sum(zm, axis=0, keepdims=True) * (1.0 / _N)
        d = (z - mean) * mask
        var = jnp.sum(d * d, axis=0, keepdims=True) * (1.0 / _N)
        hn = (z - mean) * jax.lax.rsqrt(var + _BN_EPS) * g_ref[...] + b_ref[...]
        hs = _SELU_SCALE * jnp.where(hn > 0, hn,
                                     _SELU_ALPHA * (jnp.exp(hn) - 1.0))
        y = jnp.dot(hs, w_ref[...], preferred_element_type=jnp.float32)
        o_ref[0] = y[:, :ch]
        o_ref[1] = y[:, ch:]

    return pl.pallas_call(
        body,
        out_shape=jax.ShapeDtypeStruct((2, npad, ch), jnp.float32),
    )(p, gamma, beta, w2)


def _final(p, b2):
    npad = p.shape[1]
    cdim = 2 * p.shape[2]

    def body(p_ref, b_ref, o_ref):
        z = jnp.concatenate([p_ref[0], p_ref[1]], axis=1) + b_ref[...]
        m = jnp.max(z, axis=1, keepdims=True)
        lse = jnp.log(jnp.sum(jnp.exp(z - m), axis=1, keepdims=True)) + m
        o_ref[...] = z - lse

    return pl.pallas_call(
        body,
        out_shape=jax.ShapeDtypeStruct((npad, cdim), jnp.float32),
    )(p, b2)


@jax.jit
def kernel(x, edge_index, W1, b1, gamma, beta, W2, b2):
    del b1
    n, _ = x.shape
    e = edge_index.shape[1]
    quantum = _NS * _K * _G
    e_pad = ((e + quantum - 1) // quantum) * quantum

    src = edge_index[0].astype(jnp.int32)
    dst = edge_index[1].astype(jnp.int32)
    pad_e = e_pad - e
    dump = _N + (jnp.arange(pad_e, dtype=jnp.int32) % (_NP - _N))
    src_p = jnp.concatenate([src, jnp.zeros((pad_e,), jnp.int32)]).reshape(-1, _K)
    dst_p = jnp.concatenate([dst, dump]).reshape(-1, _K)

    x_p = jnp.pad(x, ((0, _NP - n), (0, 0)))

    prop_h = _make_prop(W1.shape[1] // 2, e_pad)
    prop_c = _make_prop(W2.shape[1] // 2, e_pad)

    y1 = _mm1(x_p, W1)
    p1 = prop_h(y1, src_p, dst_p)
    p2 = prop_h(p1, src_p, dst_p)
    y2 = _mid(p2, gamma.reshape(1, -1), beta.reshape(1, -1), W2)
    p3 = prop_c(y2, src_p, dst_p)
    out = _final(p3, b2.reshape(1, -1))
    return out[:n]
